# baseline (speedup 1.0000x reference)
import functools

import jax
import jax.numpy as jnp
from jax import lax
from jax.experimental import pallas as pl
from jax.experimental.pallas import tpu as pltpu
from jax.experimental.pallas import tpu_sc as plsc

_LANES = 16


def _build_sc_kernel(num_nodes, embed_dim, batch, num_layers, num_walks,
                     slots_pad, num_workers):
  rows_per_worker = batch // num_workers
  ncg = embed_dim // _LANES
  half = rows_per_worker // 2

  mesh = plsc.VectorSubcoreMesh(core_axis_name="c", subcore_axis_name="s")

  @functools.partial(
      pl.kernel,
      mesh=mesh,
      out_type=jax.ShapeDtypeStruct((batch * embed_dim,), jnp.float32),
      scratch_types=[
          pltpu.VMEM((rows_per_worker * slots_pad,), jnp.int32),
          pltpu.VMEM((slots_pad, embed_dim), jnp.float32),
          pltpu.VMEM((slots_pad, embed_dim), jnp.float32),
          pltpu.VMEM((_LANES,), jnp.float32),
          pltpu.VMEM((embed_dim,), jnp.float32),
          pltpu.VMEM((embed_dim,), jnp.float32),
          pltpu.SemaphoreType.DMA,
          pltpu.SemaphoreType.DMA,
          pltpu.SemaphoreType.DMA,
          pltpu.SemaphoreType.DMA,
      ],
  )
  def gwarmer_kernel(emb_hbm, idx_hbm, w_hbm, out_hbm,
                     idx_v, buf0, buf1, w_v, out0, out1,
                     sem_g0, sem_g1, sem_o0, sem_o1):
    nc = 2
    wid = lax.axis_index("s") * nc + lax.axis_index("c")
    base_row = wid * rows_per_worker

    pltpu.sync_copy(
        idx_hbm.at[pl.ds(base_row * slots_pad, rows_per_worker * slots_pad)],
        idx_v)

    def shuffle(v, perm):
      return v.at[perm].get(mode="promise_in_bounds")

    def all_lanes_reduce(v, op):
      for sh in (8, 4, 2, 1):
        perm = (jnp.arange(_LANES, dtype=jnp.int32) + sh) % _LANES
        v = op(v, shuffle(v, perm))
      return v

    pltpu.sync_copy(w_hbm, w_v)
    wv = w_v[...]
    e = jnp.exp(wv - all_lanes_reduce(wv, jnp.maximum))
    probs = e / all_lanes_reduce(e, jnp.add)
    zero = jnp.zeros((_LANES,), jnp.float32)
    layer_w = [
        shuffle(probs, jnp.full((_LANES,), k, jnp.int32))
        for k in range(num_layers + 1)
    ]
    p0 = layer_w[0]
    walk_w = [w * (1.0 / num_walks) for w in layer_w[1:]]

    def gather_row(row, buf, sem):
      return pltpu.make_async_copy(
          emb_hbm.at[idx_v.at[pl.ds(row * slots_pad, slots_pad)]], buf, sem)

    def out_copy(row, out_buf, sem):
      return pltpu.make_async_copy(
          out_buf, out_hbm.at[pl.ds((base_row + row) * embed_dim, embed_dim)],
          sem)

    def compute_row(buf, out_buf):
      accs = [p0 * buf[0, pl.ds(_LANES * c, _LANES)] for c in range(ncg)]
      for k in range(num_layers):
        seg_base = 1 + num_walks * k

        def seg_body(j, carry, _seg_base=seg_base):
          return tuple(
              carry[c] + buf[_seg_base + j, pl.ds(_LANES * c, _LANES)]
              for c in range(ncg))

        seg = lax.fori_loop(0, num_walks, seg_body,
                            tuple(zero for _ in range(ncg)))
        accs = [a + walk_w[k] * s for a, s in zip(accs, seg)]
      for c in range(ncg):
        out_buf[pl.ds(_LANES * c, _LANES)] = accs[c]

    gather_row(0, buf0, sem_g0).start()

    def loop_body(r2, _):
      a = 2 * r2
      gather_row(a + 1, buf1, sem_g1).start()
      gather_row(a, buf0, sem_g0).wait()

      @pl.when(r2 > 0)
      def _():
        out_copy(0, out0, sem_o0).wait()

      compute_row(buf0, out0)
      out_copy(a, out0, sem_o0).start()

      @pl.when(r2 < half - 1)
      def _():
        gather_row(a + 2, buf0, sem_g0).start()

      gather_row(a + 1, buf1, sem_g1).wait()

      @pl.when(r2 > 0)
      def _():
        out_copy(0, out1, sem_o1).wait()

      compute_row(buf1, out1)
      out_copy(a + 1, out1, sem_o1).start()
      return 0

    lax.fori_loop(0, half, loop_body, 0)
    out_copy(0, out0, sem_o0).wait()
    out_copy(0, out1, sem_o1).wait()

  return gwarmer_kernel


def kernel(embeddings, walks, node_indices, user_weights):
  num_nodes, embed_dim = embeddings.shape
  batch, num_walks, kp1 = walks.shape
  num_layers = kp1 - 1
  num_slots = 1 + num_layers * num_walks
  slots_pad = -(-num_slots // 8) * 8
  num_workers = 32

  walk_idx = jnp.transpose(walks[:, :, 1:], (0, 2, 1)).reshape(
      batch, num_layers * num_walks)
  pad = jnp.zeros((batch, slots_pad - num_slots), jnp.int32)
  idx = jnp.concatenate([node_indices[:, None], walk_idx, pad], axis=1)
  idx_flat = idx.reshape(-1)

  w_pad = jnp.full((_LANES,), -1e30, jnp.float32).at[:kp1].set(user_weights)

  sc_kernel = _build_sc_kernel(num_nodes, embed_dim, batch, num_layers,
                               num_walks, slots_pad, num_workers)
  out_flat = sc_kernel(embeddings, idx_flat, w_pad)
  return out_flat.reshape(batch, embed_dim)
